# baseline (speedup 1.0000x reference)
import jax
import jax.numpy as jnp
from jax.experimental import pallas as pl


def _body(x_ref, adj_ref, o_ref):
    acc = jnp.dot(adj_ref[:], x_ref[:], preferred_element_type=jnp.float32)
    o_ref[:] = jnp.where(acc > 0, acc, jnp.exp(acc) - 1.0)


def kernel(x, adj):
    m, k = adj.shape
    n = x.shape[1]
    bm = 256
    return pl.pallas_call(
        _body,
        grid=(m // bm,),
        in_specs=[
            pl.BlockSpec((k, n), lambda i: (0, 0)),
            pl.BlockSpec((bm, k), lambda i: (i, 0)),
        ],
        out_specs=pl.BlockSpec((bm, n), lambda i: (i, 0)),
        out_shape=jax.ShapeDtypeStruct((m, n), jnp.float32),
    )(x, adj)
